# baseline (speedup 1.0000x reference)
import functools

import jax
import jax.numpy as jnp
from jax import lax
from jax.experimental import pallas as pl
from jax.experimental.pallas import tpu as pltpu
from jax.experimental.pallas import tpu_sc as plsc

BS = 2
S = 2304
T = 2304
K = 20
NC = 21
L = 16
NCORE = 2
NSUB = 16
GROUPS = T // NSUB // L
CHUNK = 96
NCHUNK = S // CHUNK
CAP = 128


def _sc_topk_body(corr_hbm, lbl_hbm, out_hbm,
                  lbl_vm, slab_a, slab_b, cand_vm, cm_vm, val_vm, idx_vm,
                  out_vm, sem_a, sem_b):
    b = lax.axis_index("c")
    colbase = lax.axis_index("s") * (GROUPS * L)
    pltpu.sync_copy(lbl_hbm.at[b], lbl_vm)
    iota = lax.iota(jnp.int32, L)
    neg = jnp.full((L,), -jnp.inf, jnp.float32)
    zero_i = jnp.zeros((L,), jnp.int32)

    def slab_src(g):
        return corr_hbm.at[b, :, pl.ds(colbase + g * L, L)]

    def process(slab_vm, g):
        t0 = colbase + g * L

        def chunk_body(j, _):
            base = j * CHUNK

            def rows4(r, ms):
                s0 = base + r * 4
                return (jnp.maximum(ms[0], slab_vm[s0]),
                        jnp.maximum(ms[1], slab_vm[s0 + 1]),
                        jnp.maximum(ms[2], slab_vm[s0 + 2]),
                        jnp.maximum(ms[3], slab_vm[s0 + 3]))

            ms = lax.fori_loop(0, CHUNK // 4, rows4, (neg, neg, neg, neg),
                               unroll=CHUNK // 4)
            cm_vm[j] = jnp.maximum(jnp.maximum(ms[0], ms[1]),
                                   jnp.maximum(ms[2], ms[3]))
            return 0

        lax.fori_loop(0, NCHUNK, chunk_body, 0)

        def extract(i, mprev):
            cur = neg
            for j in range(NCHUNK):
                v = cm_vm[j]
                cur = jnp.maximum(cur, jnp.where(v < mprev, v, neg))
            return cur

        th = lax.fori_loop(0, K, extract,
                           jnp.full((L,), jnp.inf, jnp.float32))

        def scan4(q, cnt):
            for u in range(4):
                s = q * 4 + u
                v = slab_vm[s]
                m = v >= th
                row = jnp.minimum(cnt, CAP - 1)
                plsc.store_scatter(cand_vm, [row, iota],
                                   jnp.full((L,), s, jnp.int32), mask=m)
                cnt = cnt + m.astype(jnp.int32)
            return cnt

        cnt = lax.fori_loop(0, S // 4, scan4, zero_i, unroll=8)
        out_vm[0] = cnt.astype(jnp.float32) + th
        pltpu.sync_copy(out_vm, out_hbm.at[b, :, pl.ds(t0, L)])

    pltpu.async_copy(slab_src(0), slab_a, sem_a)

    def pair_body(q, _):
        g0 = q * 2
        pltpu.async_copy(slab_src(g0 + 1), slab_b, sem_b)
        pltpu.make_async_copy(slab_src(g0), slab_a, sem_a).wait()
        process(slab_a, g0)
        pltpu.async_copy(slab_src(g0 + 2), slab_a, sem_a)
        pltpu.make_async_copy(slab_src(g0 + 1), slab_b, sem_b).wait()
        process(slab_b, g0 + 1)
        return 0

    lax.fori_loop(0, (GROUPS - 1) // 2, pair_body, 0)
    pltpu.make_async_copy(slab_src(GROUPS - 1), slab_a, sem_a).wait()
    process(slab_a, GROUPS - 1)


@jax.jit
def kernel(correlation_tensor, src_lbl_batch_resize):
    bs, _, h, w = correlation_tensor.shape
    corr = correlation_tensor.reshape(bs, S, T)
    lbl = src_lbl_batch_resize.reshape(bs, NC, S)
    mesh = plsc.VectorSubcoreMesh(core_axis_name="c", subcore_axis_name="s",
                                  num_cores=NCORE, num_subcores=NSUB)
    out = pl.kernel(
        _sc_topk_body,
        out_type=jax.ShapeDtypeStruct((BS, NC, T), jnp.float32),
        mesh=mesh,
        compiler_params=pltpu.CompilerParams(use_tc_tiling_on_sc=False,
                                             needs_layout_passes=False),
        scratch_types=[
            pltpu.VMEM((NC, S), jnp.float32),
            pltpu.VMEM((S, L), jnp.float32),
            pltpu.VMEM((S, L), jnp.float32),
            pltpu.VMEM((CAP, L), jnp.int32),
            pltpu.VMEM((NCHUNK, L), jnp.float32),
            pltpu.VMEM((K, L), jnp.float32),
            pltpu.VMEM((K, L), jnp.int32),
            pltpu.VMEM((NC, L), jnp.float32),
            pltpu.SemaphoreType.DMA,
            pltpu.SemaphoreType.DMA,
        ],
    )(corr, lbl)
    return out.reshape(bs, NC, h, w)


# speedup vs baseline: 1.6916x; 1.6916x over previous
import functools

import jax
import jax.numpy as jnp
from jax import lax
from jax.experimental import pallas as pl
from jax.experimental.pallas import tpu as pltpu
from jax.experimental.pallas import tpu_sc as plsc

BS = 2
S = 2304
T = 2304
K = 20
NC = 21
L = 16
NCORE = 2
NSUB = 16
T1 = 1536
TT = 768
SCT = T - T1
GROUPS = SCT // NSUB // L
CHUNK = 96
NCHUNK = S // CHUNK
CAP = 128


def _tc_topk_kernel(corr_ref, lbl_ref, out_ref):
    c = corr_ref[0]
    m = jnp.max(c, axis=0)

    def body(_, m):
        nxt = jnp.where(c < m[None, :], c, -jnp.inf)
        return jnp.max(nxt, axis=0)

    th = lax.fori_loop(0, K - 1, body, m)
    masked = jnp.where(c >= th[None, :], c, 0.0)
    out_ref[0] = lax.dot_general(
        lbl_ref[0], masked,
        dimension_numbers=(((1,), (0,)), ((), ())),
        preferred_element_type=jnp.float32,
    )


def _sc_topk_body(corr_hbm, lbl_hbm, out_hbm,
                  lbl_vm, slab_a, slab_b, cand_vm, cm_vm, val_vm, idx_vm,
                  out_vm, sem_a, sem_b):
    b = lax.axis_index("c")
    colbase = lax.axis_index("s") * (GROUPS * L)
    pltpu.sync_copy(lbl_hbm.at[b], lbl_vm)
    iota = lax.iota(jnp.int32, L)
    neg = jnp.full((L,), -jnp.inf, jnp.float32)
    zero_i = jnp.zeros((L,), jnp.int32)

    def slab_src(g):
        return corr_hbm.at[b, :, pl.ds(colbase + g * L, L)]

    def process(slab_vm, g):
        t0 = colbase + g * L

        def chunk_body(j, _):
            base = j * CHUNK

            def rows4(r, ms):
                s0 = base + r * 4
                return (jnp.maximum(ms[0], slab_vm[s0]),
                        jnp.maximum(ms[1], slab_vm[s0 + 1]),
                        jnp.maximum(ms[2], slab_vm[s0 + 2]),
                        jnp.maximum(ms[3], slab_vm[s0 + 3]))

            ms = lax.fori_loop(0, CHUNK // 4, rows4, (neg, neg, neg, neg),
                               unroll=CHUNK // 4)
            cm_vm[j] = jnp.maximum(jnp.maximum(ms[0], ms[1]),
                                   jnp.maximum(ms[2], ms[3]))
            return 0

        lax.fori_loop(0, NCHUNK, chunk_body, 0)

        def extract(i, mprev):
            cur = neg
            for j in range(NCHUNK):
                v = cm_vm[j]
                cur = jnp.maximum(cur, jnp.where(v < mprev, v, neg))
            return cur

        th = lax.fori_loop(0, K, extract,
                           jnp.full((L,), jnp.inf, jnp.float32))

        def scan4(q, cnt):
            for u in range(4):
                s = q * 4 + u
                v = slab_vm[s]
                m = v >= th
                row = jnp.minimum(cnt, CAP - 1)
                plsc.store_scatter(cand_vm, [row, iota],
                                   jnp.full((L,), s, jnp.int32), mask=m)
                cnt = cnt + m.astype(jnp.int32)
            return cnt

        cnt = lax.fori_loop(0, S // 4, scan4, zero_i, unroll=8)
        maxcnt = jnp.max(cnt)

        def ins_body(p, carry):
            r = jnp.clip(cand_vm[p], 0, S - 1)
            v = plsc.load_gather(slab_vm, [r, iota])
            v = jnp.where(p < cnt, v, neg)
            vi = r
            outv, outi = [], []
            for kk in range(K):
                rv, ri = carry[kk], carry[K + kk]
                swap = (v > rv) | ((v == rv) & (vi < ri))
                outv.append(jnp.where(swap, v, rv))
                outi.append(jnp.where(swap, vi, ri))
                v = jnp.where(swap, rv, v)
                vi = jnp.where(swap, ri, vi)
            return tuple(outv + outi)

        carry = lax.fori_loop(0, maxcnt, ins_body,
                              tuple([neg] * K + [zero_i] * K))
        for kk in range(K):
            val_vm[kk] = carry[kk]
            idx_vm[kk] = carry[K + kk]

        def comb_body(kk, acc):
            vk = val_vm[kk]
            ik = idx_vm[kk]
            return tuple(
                acc[c] + vk * plsc.load_gather(
                    lbl_vm, [jnp.full((L,), c, jnp.int32), ik])
                for c in range(NC))

        acc = lax.fori_loop(0, K, comb_body,
                            tuple([jnp.zeros((L,), jnp.float32)] * NC))
        for c in range(NC):
            out_vm[c] = acc[c]
        pltpu.sync_copy(out_vm, out_hbm.at[b, :, pl.ds(t0, L)])

    pltpu.async_copy(slab_src(0), slab_a, sem_a)

    def pair_body(q, _):
        g0 = q * 2
        pltpu.async_copy(slab_src(g0 + 1), slab_b, sem_b)
        pltpu.make_async_copy(slab_src(g0), slab_a, sem_a).wait()
        process(slab_a, g0)
        pltpu.async_copy(slab_src(g0 + 2), slab_a, sem_a)
        pltpu.make_async_copy(slab_src(g0 + 1), slab_b, sem_b).wait()
        process(slab_b, g0 + 1)
        return 0

    lax.fori_loop(0, (GROUPS - 1) // 2, pair_body, 0)
    pltpu.make_async_copy(slab_src(GROUPS - 1), slab_a, sem_a).wait()
    process(slab_a, GROUPS - 1)


def _sc_call(corr_sc, lbl):
    mesh = plsc.VectorSubcoreMesh(core_axis_name="c", subcore_axis_name="s",
                                  num_cores=NCORE, num_subcores=NSUB)
    return pl.kernel(
        _sc_topk_body,
        out_type=jax.ShapeDtypeStruct((BS, NC, SCT), jnp.float32),
        mesh=mesh,
        compiler_params=pltpu.CompilerParams(use_tc_tiling_on_sc=False,
                                             needs_layout_passes=False),
        scratch_types=[
            pltpu.VMEM((NC, S), jnp.float32),
            pltpu.VMEM((S, L), jnp.float32),
            pltpu.VMEM((S, L), jnp.float32),
            pltpu.VMEM((CAP, L), jnp.int32),
            pltpu.VMEM((NCHUNK, L), jnp.float32),
            pltpu.VMEM((K, L), jnp.float32),
            pltpu.VMEM((K, L), jnp.int32),
            pltpu.VMEM((NC, L), jnp.float32),
            pltpu.SemaphoreType.DMA,
            pltpu.SemaphoreType.DMA,
        ],
    )(corr_sc, lbl)


@jax.jit
def kernel(correlation_tensor, src_lbl_batch_resize):
    bs, _, h, w = correlation_tensor.shape
    corr = correlation_tensor.reshape(bs, S, T)
    lbl = src_lbl_batch_resize.reshape(bs, NC, S)

    corr_sc = lax.slice(corr, (0, 0, T1), (bs, S, T))
    out_sc = _sc_call(corr_sc, lbl)

    out_tc = pl.pallas_call(
        _tc_topk_kernel,
        grid=(bs, T1 // TT),
        in_specs=[
            pl.BlockSpec((1, S, TT), lambda b, t: (b, 0, t)),
            pl.BlockSpec((1, NC, S), lambda b, t: (b, 0, 0)),
        ],
        out_specs=pl.BlockSpec((1, NC, TT), lambda b, t: (b, 0, t)),
        out_shape=jax.ShapeDtypeStruct((bs, NC, T1), jnp.float32),
    )(corr, lbl)

    out = jnp.concatenate([out_tc, out_sc], axis=2)
    return out.reshape(bs, NC, h, w)


# speedup vs baseline: 1.9214x; 1.1358x over previous
import functools

import jax
import jax.numpy as jnp
from jax import lax
from jax.experimental import pallas as pl
from jax.experimental.pallas import tpu as pltpu
from jax.experimental.pallas import tpu_sc as plsc

BS = 2
S = 2304
T = 2304
K = 20
NC = 21
L = 16
NCORE = 2
NSUB = 16
T1 = 2048
TT = 512
SCT = T - T1
GROUPS = SCT // NSUB // L
CHUNK = 96
NCHUNK = S // CHUNK
CAP = 128


def _tc_topk_kernel(corr_ref, lbl_ref, out_ref):
    c = corr_ref[0]
    m = jnp.max(c, axis=0)

    def body(_, m):
        nxt = jnp.where(c < m[None, :], c, -jnp.inf)
        return jnp.max(nxt, axis=0)

    th = lax.fori_loop(0, K - 1, body, m)
    masked = jnp.where(c >= th[None, :], c, 0.0)
    out_ref[0] = lax.dot_general(
        lbl_ref[0], masked,
        dimension_numbers=(((1,), (0,)), ((), ())),
        preferred_element_type=jnp.float32,
    )


def _sc_topk_body(corr_hbm, lbl_hbm, out_hbm,
                  lbl_vm, slab_a, slab_b, cand_vm, cm_vm, val_vm, idx_vm,
                  out_vm, sem_a, sem_b):
    b = lax.axis_index("c")
    colbase = lax.axis_index("s") * (GROUPS * L)
    pltpu.sync_copy(lbl_hbm.at[b], lbl_vm)
    iota = lax.iota(jnp.int32, L)
    neg = jnp.full((L,), -jnp.inf, jnp.float32)
    zero_i = jnp.zeros((L,), jnp.int32)

    def slab_src(g):
        return corr_hbm.at[b, :, pl.ds(colbase + g * L, L)]

    def process(slab_vm, g):
        t0 = colbase + g * L

        def chunk_body(j, _):
            base = j * CHUNK

            def rows4(r, ms):
                s0 = base + r * 4
                return (jnp.maximum(ms[0], slab_vm[s0]),
                        jnp.maximum(ms[1], slab_vm[s0 + 1]),
                        jnp.maximum(ms[2], slab_vm[s0 + 2]),
                        jnp.maximum(ms[3], slab_vm[s0 + 3]))

            ms = lax.fori_loop(0, CHUNK // 4, rows4, (neg, neg, neg, neg),
                               unroll=CHUNK // 4)
            cm_vm[j] = jnp.maximum(jnp.maximum(ms[0], ms[1]),
                                   jnp.maximum(ms[2], ms[3]))
            return 0

        lax.fori_loop(0, NCHUNK, chunk_body, 0)

        def extract(i, mprev):
            cur = neg
            for j in range(NCHUNK):
                v = cm_vm[j]
                cur = jnp.maximum(cur, jnp.where(v < mprev, v, neg))
            return cur

        th = lax.fori_loop(0, K, extract,
                           jnp.full((L,), jnp.inf, jnp.float32))

        def scan4(q, cnt):
            for u in range(4):
                s = q * 4 + u
                v = slab_vm[s]
                m = v >= th
                row = jnp.minimum(cnt, CAP - 1)
                plsc.store_scatter(cand_vm, [row, iota],
                                   jnp.full((L,), s, jnp.int32), mask=m)
                cnt = cnt + m.astype(jnp.int32)
            return cnt

        cnt = lax.fori_loop(0, S // 4, scan4, zero_i, unroll=8)
        maxcnt = jnp.max(cnt)

        def ins_body(p, carry):
            r = jnp.clip(cand_vm[p], 0, S - 1)
            v = plsc.load_gather(slab_vm, [r, iota])
            v = jnp.where(p < cnt, v, neg)
            vi = r
            outv, outi = [], []
            for kk in range(K):
                rv, ri = carry[kk], carry[K + kk]
                swap = (v > rv) | ((v == rv) & (vi < ri))
                outv.append(jnp.where(swap, v, rv))
                outi.append(jnp.where(swap, vi, ri))
                v = jnp.where(swap, rv, v)
                vi = jnp.where(swap, ri, vi)
            return tuple(outv + outi)

        carry = lax.fori_loop(0, maxcnt, ins_body,
                              tuple([neg] * K + [zero_i] * K))
        for kk in range(K):
            val_vm[kk] = carry[kk]
            idx_vm[kk] = carry[K + kk]

        def comb_body(kk, acc):
            vk = val_vm[kk]
            ik = idx_vm[kk]
            return tuple(
                acc[c] + vk * plsc.load_gather(
                    lbl_vm, [jnp.full((L,), c, jnp.int32), ik])
                for c in range(NC))

        acc = lax.fori_loop(0, K, comb_body,
                            tuple([jnp.zeros((L,), jnp.float32)] * NC))
        for c in range(NC):
            out_vm[c] = acc[c]
        pltpu.sync_copy(out_vm, out_hbm.at[b, :, pl.ds(t0, L)])

    pltpu.async_copy(slab_src(0), slab_a, sem_a)

    def pair_body(q, _):
        g0 = q * 2
        pltpu.async_copy(slab_src(g0 + 1), slab_b, sem_b)
        pltpu.make_async_copy(slab_src(g0), slab_a, sem_a).wait()
        process(slab_a, g0)
        pltpu.async_copy(slab_src(g0 + 2), slab_a, sem_a)
        pltpu.make_async_copy(slab_src(g0 + 1), slab_b, sem_b).wait()
        process(slab_b, g0 + 1)
        return 0

    lax.fori_loop(0, (GROUPS - 1) // 2, pair_body, 0)
    pltpu.make_async_copy(slab_src(GROUPS - 1), slab_a, sem_a).wait()
    process(slab_a, GROUPS - 1)


def _sc_call(corr_sc, lbl):
    mesh = plsc.VectorSubcoreMesh(core_axis_name="c", subcore_axis_name="s",
                                  num_cores=NCORE, num_subcores=NSUB)
    return pl.kernel(
        _sc_topk_body,
        out_type=jax.ShapeDtypeStruct((BS, NC, SCT), jnp.float32),
        mesh=mesh,
        compiler_params=pltpu.CompilerParams(use_tc_tiling_on_sc=False,
                                             needs_layout_passes=False),
        scratch_types=[
            pltpu.VMEM((NC, S), jnp.float32),
            pltpu.VMEM((S, L), jnp.float32),
            pltpu.VMEM((S, L), jnp.float32),
            pltpu.VMEM((CAP, L), jnp.int32),
            pltpu.VMEM((NCHUNK, L), jnp.float32),
            pltpu.VMEM((K, L), jnp.float32),
            pltpu.VMEM((K, L), jnp.int32),
            pltpu.VMEM((NC, L), jnp.float32),
            pltpu.SemaphoreType.DMA,
            pltpu.SemaphoreType.DMA,
        ],
    )(corr_sc, lbl)


@jax.jit
def kernel(correlation_tensor, src_lbl_batch_resize):
    bs, _, h, w = correlation_tensor.shape
    corr = correlation_tensor.reshape(bs, S, T)
    lbl = src_lbl_batch_resize.reshape(bs, NC, S)

    corr_sc = lax.slice(corr, (0, 0, T1), (bs, S, T))
    out_sc = _sc_call(corr_sc, lbl)

    out_tc = pl.pallas_call(
        _tc_topk_kernel,
        grid=(bs, T1 // TT),
        in_specs=[
            pl.BlockSpec((1, S, TT), lambda b, t: (b, 0, t)),
            pl.BlockSpec((1, NC, S), lambda b, t: (b, 0, 0)),
        ],
        out_specs=pl.BlockSpec((1, NC, TT), lambda b, t: (b, 0, t)),
        out_shape=jax.ShapeDtypeStruct((bs, NC, T1), jnp.float32),
    )(corr, lbl)

    out = jnp.concatenate([out_tc, out_sc], axis=2)
    return out.reshape(bs, NC, h, w)
